# Initial kernel scaffold; baseline (speedup 1.0000x reference)
import jax
import jax.numpy as jnp
from jax.experimental import pallas as pl


def kernel(solute_adj, solute_meth, solvent_meth, solvent_adj_meth, smiles, params):
    raise NotImplementedError("write your pallas kernel here")


# speedup vs baseline: 2.6251x; 2.6251x over previous
import functools
import math

import jax
import jax.numpy as jnp
from jax.experimental import pallas as pl
from jax.experimental.pallas import tpu as pltpu

B = 4
N_SOLUTE = 2076
N_SOLVENT = 16335
NFEAT = 128
NHID = 64
NCLASS = 100


def _mm_body(x_ref, w_ref, b_ref, o_ref):
    o_ref[...] = (
        jnp.dot(x_ref[...], w_ref[...], preferred_element_type=jnp.float32)
        + b_ref[...]
    )


def _matmul_bias(x, W, b, block_rows=512):
    n, din = x.shape
    dout = W.shape[0]
    n_pad = ((n + block_rows - 1) // block_rows) * block_rows
    if n_pad != n:
        x = jnp.pad(x, ((0, n_pad - n), (0, 0)))
    wt = W.T
    b2 = b.reshape(1, dout)
    out = pl.pallas_call(
        _mm_body,
        grid=(n_pad // block_rows,),
        in_specs=[
            pl.BlockSpec((block_rows, din), lambda i: (i, 0)),
            pl.BlockSpec((din, dout), lambda i: (0, 0)),
            pl.BlockSpec((1, dout), lambda i: (0, 0)),
        ],
        out_specs=pl.BlockSpec((block_rows, dout), lambda i: (i, 0)),
        out_shape=jax.ShapeDtypeStruct((n_pad, dout), jnp.float32),
    )(x, wt, b2)
    return out[:n]


def _gru_mean(x, p):
    def step(Wih, Whh, bih, bhh):
        gi = x @ Wih.T + bih
        gh = bhh
        i_r, i_z, i_n = jnp.split(gi, 3, axis=-1)
        h_r, h_z, h_n = jnp.split(gh, 3, axis=-1)
        r = jax.nn.sigmoid(i_r + h_r)
        z = jax.nn.sigmoid(i_z + h_z)
        n = jnp.tanh(i_n + r * h_n)
        return (1.0 - z) * n
    out = jnp.concatenate([
        step(p['gru_Wih_f'], p['gru_Whh_f'], p['gru_bih_f'], p['gru_bhh_f']),
        step(p['gru_Wih_b'], p['gru_Whh_b'], p['gru_bih_b'], p['gru_bhh_b']),
    ], axis=-1)
    out = jax.nn.relu(out)
    return jnp.mean(out, axis=0, keepdims=True)


def _tconv(x, edge_index, p, name):
    src = edge_index[0]
    dst = edge_index[1]
    dout = p[name + '_Wq'].shape[0]
    din = x.shape[1]
    n = x.shape[0]
    Wcat = jnp.concatenate(
        [p[name + '_Wq'], p[name + '_Wk'], p[name + '_Wv'], p[name + '_Ws']], axis=0)
    bcat = jnp.concatenate(
        [p[name + '_bq'], p[name + '_bk'], p[name + '_bv'], p[name + '_bs']], axis=0)
    proj = _matmul_bias(x, Wcat, bcat)
    q = proj[:, 0 * dout:1 * dout]
    k = proj[:, 1 * dout:2 * dout]
    v = proj[:, 2 * dout:3 * dout]
    skip = proj[:, 3 * dout:4 * dout]
    e = jnp.sum(q[dst] * k[src], axis=-1) / math.sqrt(dout)
    ex = jnp.exp(e - jnp.max(e))
    denom = jax.ops.segment_sum(ex, dst, num_segments=n)
    agg = jax.ops.segment_sum(ex[:, None] * v[src], dst, num_segments=n)
    agg = agg / (denom[:, None] + 1e-16)
    return agg + skip


def _set2set(x, p):
    nper = x.shape[0] // B
    d = x.shape[-1]
    xb = x.reshape(B, nper, d)
    q_star = jnp.zeros((B, 2 * d), dtype=x.dtype)
    h = jnp.zeros((B, d), dtype=x.dtype)
    c = jnp.zeros((B, d), dtype=x.dtype)
    for _ in range(2):
        g = q_star @ p['lstm_Wih'].T + p['lstm_bih'] + h @ p['lstm_Whh'].T + p['lstm_bhh']
        ii, ff, gg, oo = jnp.split(g, 4, axis=-1)
        ii = jax.nn.sigmoid(ii)
        ff = jax.nn.sigmoid(ff)
        gg = jnp.tanh(gg)
        oo = jax.nn.sigmoid(oo)
        c = ff * c + ii * gg
        h = oo * jnp.tanh(c)
        q = h
        e = jnp.einsum('bnd,bd->bn', xb, q)
        emax = jnp.max(e, axis=1, keepdims=True)
        ex = jnp.exp(e - emax)
        a = ex / (jnp.sum(ex, axis=1, keepdims=True) + 1e-16)
        r = jnp.einsum('bn,bnd->bd', a, xb)
        q_star = jnp.concatenate([q, r], axis=-1)
    return q_star


def kernel(solute_adj, solute_meth, solvent_meth, solvent_adj_meth, smiles, params):
    p = params
    solute_smile = smiles[0]
    meth_solvent = smiles[5]
    sv = jnp.take(p['embed'], solute_smile, axis=0)
    mv = jnp.take(p['embed'], meth_solvent, axis=0)
    after_solute = jnp.tile(_gru_mean(sv, p), (B, 1))
    after_meth = jnp.tile(_gru_mean(mv, p), (B, 1))

    xs0 = solute_meth.reshape(B * N_SOLUTE, NFEAT)
    xv0 = solvent_meth.reshape(B * N_SOLVENT, NFEAT)
    init_s = _matmul_bias(xs0, p['fc1_W'], p['fc1_b'])
    init_v = _matmul_bias(xv0, p['fc1_W'], p['fc1_b'])

    xs = jax.nn.relu(_tconv(xs0, solute_adj, p, 'c1'))
    xs = _tconv(xs, solute_adj, p, 'c2') + init_s
    xv = jax.nn.relu(_tconv(xv0, solvent_adj_meth, p, 'c1'))
    xv = _tconv(xv, solvent_adj_meth, p, 'c2') + init_v

    ss = _set2set(xs, p)
    vv = _set2set(xv, p)
    data = jnp.concatenate([ss, after_solute, vv, after_meth], axis=1)
    data = jax.nn.relu(data @ p['fc2_W'].T + p['fc2_b'])
    data = jax.nn.relu(data @ p['fc3_W'].T + p['fc3_b'])
    data = jax.nn.relu(data @ p['fc4_W'].T + p['fc4_b'])
    return data @ p['fc5_W'].T + p['fc5_b']
